# Initial kernel scaffold; baseline (speedup 1.0000x reference)
import jax
import jax.numpy as jnp
from jax.experimental import pallas as pl


def kernel(x, edge_index, kernel_id, W, gamma, beta):
    raise NotImplementedError("write your pallas kernel here")


# speedup vs baseline: 53.4180x; 53.4180x over previous
import functools

import jax
import jax.numpy as jnp
from jax import lax
from jax.experimental import pallas as pl
from jax.experimental.pallas import tpu as pltpu
from jax.experimental.pallas import tpu_sc as plsc


def _xw_body(n_nodes, x_ref, w_ref, kid_ref, src_ref, xw_ref, gidx_ref):
    xw_ref[0] = jnp.dot(x_ref[...], w_ref[0],
                        preferred_element_type=jnp.float32)

    @pl.when(pl.program_id(0) == 0)
    def _():
        gidx_ref[...] = kid_ref[...] * n_nodes + src_ref[...]


def _make_xw_call(K, N, C, ER):
    return pl.pallas_call(
        functools.partial(_xw_body, N),
        grid=(K,),
        in_specs=[
            pl.BlockSpec((N, C), lambda k: (0, 0)),
            pl.BlockSpec((1, C, C), lambda k: (k, 0, 0)),
            pl.BlockSpec((ER, 128), lambda k: (0, 0)),
            pl.BlockSpec((ER, 128), lambda k: (0, 0)),
        ],
        out_specs=[
            pl.BlockSpec((1, N, C), lambda k: (k, 0, 0)),
            pl.BlockSpec((ER, 128), lambda k: (0, 0)),
        ],
        out_shape=[
            jax.ShapeDtypeStruct((K, N, C), jnp.float32),
            jax.ShapeDtypeStruct((ER, 128), jnp.int32),
        ],
    )


def _make_sc_call(N, C, E, B):
    NW = 32
    epw = E // NW
    nch = epw // B
    sr = 640
    sr_last = N - 15 * sr

    mesh = plsc.VectorSubcoreMesh(core_axis_name="c", subcore_axis_name="s")

    @functools.partial(
        pl.kernel,
        out_type=jax.ShapeDtypeStruct((2, N, C), jnp.float32),
        mesh=mesh,
        scratch_types=[
            pltpu.VMEM_SHARED((N, C), jnp.float32),
            pltpu.VMEM((B,), jnp.int32),
            pltpu.VMEM((B,), jnp.int32),
            pltpu.VMEM((B, C), jnp.float32),
        ],
    )
    def sc_call(xw_hbm, gidx_hbm, dst_hbm, zeros_hbm, out_hbm,
                slab, gbuf, dbuf, rows):
        c = lax.axis_index("c")
        s = lax.axis_index("s")
        w = c * 16 + s
        ebase = w * epw

        @pl.when(s < 15)
        def _():
            pltpu.sync_copy(zeros_hbm.at[pl.ds(s * sr, sr)],
                            slab.at[pl.ds(s * sr, sr)])

        @pl.when(s == 15)
        def _():
            pltpu.sync_copy(zeros_hbm.at[pl.ds(15 * sr, sr_last)],
                            slab.at[pl.ds(15 * sr, sr_last)])

        plsc.subcore_barrier()

        def body(g, carry):
            base = pl.multiple_of(ebase + g * B, 8)
            pltpu.sync_copy(gidx_hbm.at[pl.ds(base, B)], gbuf)
            pltpu.sync_copy(dst_hbm.at[pl.ds(base, B)], dbuf)
            pltpu.sync_copy(xw_hbm.at[gbuf], rows)
            pltpu.sync_copy(rows, slab.at[dbuf], add=True)
            return carry

        lax.fori_loop(0, nch, body, 0)

        plsc.subcore_barrier()

        @pl.when(s < 15)
        def _():
            pltpu.sync_copy(slab.at[pl.ds(s * sr, sr)],
                            out_hbm.at[c].at[pl.ds(s * sr, sr)])

        @pl.when(s == 15)
        def _():
            pltpu.sync_copy(slab.at[pl.ds(15 * sr, sr_last)],
                            out_hbm.at[c].at[pl.ds(15 * sr, sr_last)])

    return sc_call


def _bn_body(n_nodes, p_ref, gamma_ref, beta_ref, o_ref):
    out = p_ref[0] + p_ref[1]
    mean = jnp.mean(out, axis=0, keepdims=True)
    ctr = out - mean
    var = jnp.mean(ctr * ctr, axis=0, keepdims=True)
    y = ctr * lax.rsqrt(var + 1e-5) * gamma_ref[...] + beta_ref[...]
    o_ref[...] = jnp.maximum(y, 0.0)


def _make_bn_call(N, C):
    return pl.pallas_call(
        functools.partial(_bn_body, N),
        in_specs=[
            pl.BlockSpec((2, N, C), lambda: (0, 0, 0)),
            pl.BlockSpec((1, C), lambda: (0, 0)),
            pl.BlockSpec((1, C), lambda: (0, 0)),
        ],
        out_specs=pl.BlockSpec((N, C), lambda: (0, 0)),
        out_shape=jax.ShapeDtypeStruct((N, C), jnp.float32),
    )


@jax.jit
def kernel(x, edge_index, kernel_id, W, gamma, beta):
    N, C = x.shape
    K = W.shape[0]
    E = kernel_id.shape[0]
    B = 80

    src2d = edge_index[0].astype(jnp.int32).reshape(E // 128, 128)
    kid2d = kernel_id.astype(jnp.int32).reshape(E // 128, 128)
    dst = edge_index[1].astype(jnp.int32)

    xw, gidx2d = _make_xw_call(K, N, C, E // 128)(
        x, W, kid2d, src2d)

    zeros = jnp.zeros((N, C), jnp.float32)
    partials = _make_sc_call(N, C, E, B)(
        xw.reshape(K * N, C), gidx2d.reshape(E), dst, zeros)

    out = _make_bn_call(N, C)(
        partials, gamma.reshape(1, C), beta.reshape(1, C))
    return out


# speedup vs baseline: 115.7033x; 2.1660x over previous
import functools

import jax
import jax.numpy as jnp
from jax import lax
from jax.experimental import pallas as pl
from jax.experimental.pallas import tpu as pltpu
from jax.experimental.pallas import tpu_sc as plsc


def _xw_body(n_nodes, x_ref, w_ref, kid_ref, src_ref, xw_ref, gidx_ref):
    xw_ref[0] = jnp.dot(x_ref[...], w_ref[0],
                        preferred_element_type=jnp.float32)

    @pl.when(pl.program_id(0) == 0)
    def _():
        gidx_ref[...] = kid_ref[...] * n_nodes + src_ref[...]


def _make_xw_call(K, N, C, ER):
    return pl.pallas_call(
        functools.partial(_xw_body, N),
        grid=(K,),
        in_specs=[
            pl.BlockSpec((N, C), lambda k: (0, 0)),
            pl.BlockSpec((1, C, C), lambda k: (k, 0, 0)),
            pl.BlockSpec((ER, 128), lambda k: (0, 0)),
            pl.BlockSpec((ER, 128), lambda k: (0, 0)),
        ],
        out_specs=[
            pl.BlockSpec((1, N, C), lambda k: (k, 0, 0)),
            pl.BlockSpec((ER, 128), lambda k: (0, 0)),
        ],
        out_shape=[
            jax.ShapeDtypeStruct((K, N, C), jnp.float32),
            jax.ShapeDtypeStruct((ER, 128), jnp.int32),
        ],
    )


def _make_sc_call(N, C, E, B):
    NW = 32
    epw = E // NW
    nch = epw // B
    DEPTH = 4
    H = 2
    ngo = (nch + DEPTH - 1) // DEPTH
    sr = 640
    sr_last = N - 15 * sr

    mesh = plsc.VectorSubcoreMesh(core_axis_name="c", subcore_axis_name="s")

    @functools.partial(
        pl.kernel,
        out_type=jax.ShapeDtypeStruct((2, N, C), jnp.float32),
        mesh=mesh,
        scratch_types=[
            pltpu.VMEM_SHARED((N, C), jnp.float32),
            [pltpu.VMEM((B,), jnp.int32)] * DEPTH,
            [pltpu.VMEM((B,), jnp.int32)] * DEPTH,
            [pltpu.VMEM((B, C), jnp.float32)] * DEPTH,
            [pltpu.SemaphoreType.DMA] * DEPTH,
            [pltpu.SemaphoreType.DMA] * DEPTH,
            [pltpu.SemaphoreType.DMA] * DEPTH,
        ],
    )
    def sc_call(xw_hbm, gidx_hbm, dst_hbm, zeros_hbm, out_hbm,
                slab, gbufs, dbufs, rows, isems, dsems, gsems):
        c = lax.axis_index("c")
        s = lax.axis_index("s")
        w = c * 16 + s
        ebase = w * epw

        def stage_idx(q, d):
            base = pl.multiple_of(ebase + q * B, 8)
            pltpu.async_copy(gidx_hbm.at[pl.ds(base, B)], gbufs[d], isems[d])
            pltpu.async_copy(dst_hbm.at[pl.ds(base, B)], dbufs[d], dsems[d])

        def wait_idx(d):
            pltpu.make_async_copy(gidx_hbm.at[pl.ds(0, B)],
                                  gbufs[d], isems[d]).wait()
            pltpu.make_async_copy(dst_hbm.at[pl.ds(0, B)],
                                  dbufs[d], dsems[d]).wait()

        @pl.when(s < 15)
        def _():
            pltpu.sync_copy(zeros_hbm.at[pl.ds(s * sr, sr)],
                            slab.at[pl.ds(s * sr, sr)])

        @pl.when(s == 15)
        def _():
            pltpu.sync_copy(zeros_hbm.at[pl.ds(15 * sr, sr_last)],
                            slab.at[pl.ds(15 * sr, sr_last)])

        for q in range(DEPTH):
            stage_idx(q, q)
        plsc.subcore_barrier()
        for q in range(H):
            wait_idx(q)
            pltpu.async_copy(xw_hbm.at[gbufs[q]], rows[q], gsems[q])

        def body(go, carry):
            for d in range(DEPTH):
                g = go * DEPTH + d
                dh = (d + H) % DEPTH

                @pl.when(g + H < nch)
                def _():
                    wait_idx(dh)
                    pltpu.async_copy(xw_hbm.at[gbufs[dh]], rows[dh],
                                     gsems[dh])

                @pl.when(g < nch)
                def _():
                    pltpu.make_async_copy(xw_hbm.at[gbufs[d]], rows[d],
                                          gsems[d]).wait()
                    pltpu.sync_copy(rows[d], slab.at[dbufs[d]], add=True)

                @pl.when(g + DEPTH < nch)
                def _():
                    stage_idx(g + DEPTH, d)
            return carry

        lax.fori_loop(0, ngo, body, 0)

        plsc.subcore_barrier()

        @pl.when(s < 15)
        def _():
            pltpu.sync_copy(slab.at[pl.ds(s * sr, sr)],
                            out_hbm.at[c].at[pl.ds(s * sr, sr)])

        @pl.when(s == 15)
        def _():
            pltpu.sync_copy(slab.at[pl.ds(15 * sr, sr_last)],
                            out_hbm.at[c].at[pl.ds(15 * sr, sr_last)])

    return sc_call


def _bn_body(n_nodes, p_ref, gamma_ref, beta_ref, o_ref):
    out = p_ref[0] + p_ref[1]
    mean = jnp.mean(out, axis=0, keepdims=True)
    ctr = out - mean
    var = jnp.mean(ctr * ctr, axis=0, keepdims=True)
    y = ctr * lax.rsqrt(var + 1e-5) * gamma_ref[...] + beta_ref[...]
    o_ref[...] = jnp.maximum(y, 0.0)


def _make_bn_call(N, C):
    return pl.pallas_call(
        functools.partial(_bn_body, N),
        in_specs=[
            pl.BlockSpec((2, N, C), lambda: (0, 0, 0)),
            pl.BlockSpec((1, C), lambda: (0, 0)),
            pl.BlockSpec((1, C), lambda: (0, 0)),
        ],
        out_specs=pl.BlockSpec((N, C), lambda: (0, 0)),
        out_shape=jax.ShapeDtypeStruct((N, C), jnp.float32),
    )


@jax.jit
def kernel(x, edge_index, kernel_id, W, gamma, beta):
    N, C = x.shape
    K = W.shape[0]
    E = kernel_id.shape[0]
    B = 80

    src2d = edge_index[0].astype(jnp.int32).reshape(E // 128, 128)
    kid2d = kernel_id.astype(jnp.int32).reshape(E // 128, 128)
    dst = edge_index[1].astype(jnp.int32)

    xw, gidx2d = _make_xw_call(K, N, C, E // 128)(
        x, W, kid2d, src2d)

    zeros = jnp.zeros((N, C), jnp.float32)
    partials = _make_sc_call(N, C, E, B)(
        xw.reshape(K * N, C), gidx2d.reshape(E), dst, zeros)

    out = _make_bn_call(N, C)(
        partials, gamma.reshape(1, C), beta.reshape(1, C))
    return out
